# baseline (speedup 1.0000x reference)
import functools

import jax
import jax.numpy as jnp
from jax import lax
from jax.experimental import pallas as pl
from jax.experimental.pallas import tpu as pltpu
from jax.experimental.pallas import tpu_sc as plsc

B = 16384
D = 32
IDXC = 128


def _build_sc_gather(nc, ns):
    nw = nc * ns
    bpw = B // nw
    chunks = bpw // IDXC
    mesh = plsc.VectorSubcoreMesh(core_axis_name="c", subcore_axis_name="s")

    @functools.partial(
        pl.kernel,
        mesh=mesh,
        compiler_params=pltpu.CompilerParams(use_tc_tiling_on_sc=False),
        out_type=[jax.ShapeDtypeStruct((B, D), jnp.float32)] * 4,
        scratch_types=[
            pltpu.VMEM((chunks, IDXC), jnp.int32),
            pltpu.VMEM((chunks, IDXC), jnp.int32),
            pltpu.VMEM((bpw, D), jnp.float32),
            pltpu.VMEM((bpw, D), jnp.float32),
            pltpu.VMEM((bpw, D), jnp.float32),
            pltpu.VMEM((bpw, D), jnp.float32),
            pltpu.SemaphoreType.DMA,
        ],
    )
    def gather_kernel(uidx_hbm, iidx_hbm, ug_hbm, ig_hbm, um_hbm, im_hbm,
                      oug, oig, oum, oim,
                      uidx_v, iidx_v, ug_v, ig_v, um_v, im_v, sem):
        wid = lax.axis_index("s") * nc + lax.axis_index("c")
        base = wid * bpw
        pltpu.sync_copy(uidx_hbm.at[pl.ds(wid * chunks, chunks)], uidx_v)
        pltpu.sync_copy(iidx_hbm.at[pl.ds(wid * chunks, chunks)], iidx_v)
        copies = []
        for j in range(chunks):
            sl = pl.ds(j * IDXC, IDXC)
            copies.append(pltpu.async_copy(ug_hbm.at[uidx_v.at[j]], ug_v.at[sl], sem))
            copies.append(pltpu.async_copy(ig_hbm.at[iidx_v.at[j]], ig_v.at[sl], sem))
            copies.append(pltpu.async_copy(um_hbm.at[uidx_v.at[j]], um_v.at[sl], sem))
            copies.append(pltpu.async_copy(im_hbm.at[iidx_v.at[j]], im_v.at[sl], sem))
        for c in copies:
            c.wait()
        pltpu.sync_copy(ug_v, oug.at[pl.ds(base, bpw)])
        pltpu.sync_copy(ig_v, oig.at[pl.ds(base, bpw)])
        pltpu.sync_copy(um_v, oum.at[pl.ds(base, bpw)])
        pltpu.sync_copy(im_v, oim.at[pl.ds(base, bpw)])

    return gather_kernel


BM = 2048


def _dense_body(ug, ig, um, im, w0u, w0i, b0, w1, b1, w2, b2, wpg, wpm, bp, out):
    h = jnp.dot(um[...], w0u[...], preferred_element_type=jnp.float32)
    h = h + jnp.dot(im[...], w0i[...], preferred_element_type=jnp.float32)
    h = jnp.maximum(h + b0[...], 0.0)
    h = jnp.maximum(jnp.dot(h, w1[...], preferred_element_type=jnp.float32) + b1[...], 0.0)
    h = jnp.maximum(jnp.dot(h, w2[...], preferred_element_type=jnp.float32) + b2[...], 0.0)
    g = ug[...] * ig[...]
    logit = (jnp.sum(g * wpg[...], axis=1, keepdims=True)
             + jnp.sum(h * wpm[...], axis=1, keepdims=True) + bp[...])
    out[...] = 1.0 / (1.0 + jnp.exp(-logit))


def _dense_tc(ug, ig, um, im, w0u, w0i, b0, w1, b1, w2, b2, wpg, wpm, bp):
    row = lambda i: (i, 0)
    rep = lambda i: (0, 0)
    h0, h1, h2 = b0.shape[1], b1.shape[1], b2.shape[1]
    return pl.pallas_call(
        _dense_body,
        grid=(B // BM,),
        in_specs=[
            pl.BlockSpec((BM, D), row),
            pl.BlockSpec((BM, D), row),
            pl.BlockSpec((BM, D), row),
            pl.BlockSpec((BM, D), row),
            pl.BlockSpec((D, h0), rep),
            pl.BlockSpec((D, h0), rep),
            pl.BlockSpec((1, h0), rep),
            pl.BlockSpec((h0, h1), rep),
            pl.BlockSpec((1, h1), rep),
            pl.BlockSpec((h1, h2), rep),
            pl.BlockSpec((1, h2), rep),
            pl.BlockSpec((1, D), rep),
            pl.BlockSpec((1, h2), rep),
            pl.BlockSpec((1, 1), rep),
        ],
        out_specs=pl.BlockSpec((BM, 1), row),
        out_shape=jax.ShapeDtypeStruct((B, 1), jnp.float32),
    )(ug, ig, um, im, w0u, w0i, b0, w1, b1, w2, b2, wpg, wpm, bp)


def kernel(user_indices, item_indices, ue_gmf, ie_gmf, ue_mlp, ie_mlp,
           W0, b0, W1, b1, W2, b2, Wp, bp):
    info = plsc.get_sparse_core_info()
    gather = _build_sc_gather(info.num_cores, info.num_subcores)
    uidx = user_indices.astype(jnp.int32).reshape(B // IDXC, IDXC)
    iidx = item_indices.astype(jnp.int32).reshape(B // IDXC, IDXC)
    ug, ig, um, im = gather(uidx, iidx, ue_gmf, ie_gmf, ue_mlp, ie_mlp)
    h2 = W2.shape[1]
    out = _dense_tc(
        ug, ig, um, im,
        W0[:D], W0[D:], b0.reshape(1, -1),
        W1, b1.reshape(1, -1),
        W2, b2.reshape(1, -1),
        Wp[:D].reshape(1, D), Wp[D:].reshape(1, h2), bp.reshape(1, 1),
    )
    return out.reshape(-1)


# speedup vs baseline: 1.5677x; 1.5677x over previous
import functools

import jax
import jax.numpy as jnp
from jax import lax
from jax.experimental import pallas as pl
from jax.experimental.pallas import tpu as pltpu
from jax.experimental.pallas import tpu_sc as plsc

B = 16384
D = 32
BN = 4096
G4 = BN // 4
NV = 1000000
NBLK = (NV + BN - 1) // BN
N4 = NBLK * G4
GCH = 128


def _repack_body(ta_ref, tb_ref, outa_ref, outb_ref):
    i = pl.program_id(0)
    cols = jax.lax.broadcasted_iota(jnp.int32, (D, BN), 1) + i * BN
    mask = cols < NV
    qa = jnp.transpose(jnp.where(mask, ta_ref[...], 0.0))
    qb = jnp.transpose(jnp.where(mask, tb_ref[...], 0.0))
    for j in range(4):
        outa_ref[:, j * D:(j + 1) * D] = qa[j * G4:(j + 1) * G4, :]
        outb_ref[:, j * D:(j + 1) * D] = qb[j * G4:(j + 1) * G4, :]


def _repack_tc(ta_t, tb_t):
    return pl.pallas_call(
        _repack_body,
        grid=(NBLK,),
        in_specs=[
            pl.BlockSpec((D, BN), lambda i: (0, i)),
            pl.BlockSpec((D, BN), lambda i: (0, i)),
        ],
        out_specs=[
            pl.BlockSpec((G4, 128), lambda i: (i, 0)),
            pl.BlockSpec((G4, 128), lambda i: (i, 0)),
        ],
        out_shape=[jax.ShapeDtypeStruct((N4, 128), jnp.float32)] * 2,
    )(ta_t, tb_t)


def _build_sc_gather(nc, ns):
    nw = nc * ns
    bpw = B // nw
    nch = bpw // GCH
    mesh = plsc.VectorSubcoreMesh(core_axis_name="c", subcore_axis_name="s")

    @functools.partial(
        pl.kernel,
        mesh=mesh,
        compiler_params=pltpu.CompilerParams(use_tc_tiling_on_sc=False),
        out_type=[jax.ShapeDtypeStruct((B, 128), jnp.float32)] * 4,
        scratch_types=[
            pltpu.VMEM((nch, GCH), jnp.int32),
            pltpu.VMEM((nch, GCH), jnp.int32),
            pltpu.VMEM((GCH, 128), jnp.float32),
            pltpu.VMEM((GCH, 128), jnp.float32),
            pltpu.SemaphoreType.DMA,
            pltpu.SemaphoreType.DMA,
        ],
    )
    def gather_kernel(uidx_hbm, iidx_hbm, ug_hbm, ig_hbm, um_hbm, im_hbm,
                      oug, oig, oum, oim,
                      uidx_v, iidx_v, rows_a, rows_b, sem_a, sem_b):
        wid = lax.axis_index("s") * nc + lax.axis_index("c")
        base = wid * bpw
        pltpu.sync_copy(uidx_hbm.at[pl.ds(wid * nch, nch)], uidx_v)
        pltpu.sync_copy(iidx_hbm.at[pl.ds(wid * nch, nch)], iidx_v)

        seq = []
        for tab, idxv, outh in ((ug_hbm, uidx_v, oug), (ig_hbm, iidx_v, oig),
                                (um_hbm, uidx_v, oum), (im_hbm, iidx_v, oim)):
            for cc in range(nch):
                seq.append((tab, idxv, outh, cc))

        bufs = (rows_a, rows_b)
        sems = (sem_a, sem_b)

        def start(s):
            tab, idxv, _, cc = seq[s]
            pltpu.async_copy(tab.at[idxv.at[cc]], bufs[s % 2], sems[s % 2])

        start(0)
        for s in range(len(seq)):
            tab, idxv, outh, cc = seq[s]
            p = s % 2
            if s + 1 < len(seq):
                start(s + 1)
            pltpu.make_async_copy(tab.at[idxv.at[cc]], bufs[p], sems[p]).wait()
            pltpu.sync_copy(bufs[p], outh.at[pl.ds(base + cc * GCH, GCH)])

    return gather_kernel


BM = 2048


def _dense_body(ugp, igp, ump, imp, ju, ji, w0u, w0i, b0, w1, b1, w2, b2,
                wpg, wpm, bp, out):
    sel_u = ju[...]
    sel_i = ji[...]
    ug = jnp.zeros((BM, D), jnp.float32)
    um = jnp.zeros((BM, D), jnp.float32)
    ig = jnp.zeros((BM, D), jnp.float32)
    im = jnp.zeros((BM, D), jnp.float32)
    for j in range(4):
        mu = (sel_u == j).astype(jnp.float32)
        mi = (sel_i == j).astype(jnp.float32)
        ug = ug + ugp[:, j * D:(j + 1) * D] * mu
        um = um + ump[:, j * D:(j + 1) * D] * mu
        ig = ig + igp[:, j * D:(j + 1) * D] * mi
        im = im + imp[:, j * D:(j + 1) * D] * mi
    h = jnp.dot(um, w0u[...], preferred_element_type=jnp.float32)
    h = h + jnp.dot(im, w0i[...], preferred_element_type=jnp.float32)
    h = jnp.maximum(h + b0[...], 0.0)
    h = jnp.maximum(jnp.dot(h, w1[...], preferred_element_type=jnp.float32) + b1[...], 0.0)
    h = jnp.maximum(jnp.dot(h, w2[...], preferred_element_type=jnp.float32) + b2[...], 0.0)
    g = ug * ig
    logit = (jnp.sum(g * wpg[...], axis=1, keepdims=True)
             + jnp.sum(h * wpm[...], axis=1, keepdims=True) + bp[...])
    out[...] = 1.0 / (1.0 + jnp.exp(-logit))


def _dense_tc(ugp, igp, ump, imp, ju, ji,
              w0u, w0i, b0, w1, b1, w2, b2, wpg, wpm, bp):
    row = lambda i: (i, 0)
    rep = lambda i: (0, 0)
    h0, h1, h2 = b0.shape[1], b1.shape[1], b2.shape[1]
    return pl.pallas_call(
        _dense_body,
        grid=(B // BM,),
        in_specs=[
            pl.BlockSpec((BM, 128), row),
            pl.BlockSpec((BM, 128), row),
            pl.BlockSpec((BM, 128), row),
            pl.BlockSpec((BM, 128), row),
            pl.BlockSpec((BM, 1), row),
            pl.BlockSpec((BM, 1), row),
            pl.BlockSpec((D, h0), rep),
            pl.BlockSpec((D, h0), rep),
            pl.BlockSpec((1, h0), rep),
            pl.BlockSpec((h0, h1), rep),
            pl.BlockSpec((1, h1), rep),
            pl.BlockSpec((h1, h2), rep),
            pl.BlockSpec((1, h2), rep),
            pl.BlockSpec((1, D), rep),
            pl.BlockSpec((1, h2), rep),
            pl.BlockSpec((1, 1), rep),
        ],
        out_specs=pl.BlockSpec((BM, 1), row),
        out_shape=jax.ShapeDtypeStruct((B, 1), jnp.float32),
    )(ugp, igp, ump, imp, ju, ji, w0u, w0i, b0, w1, b1, w2, b2, wpg, wpm, bp)


def kernel(user_indices, item_indices, ue_gmf, ie_gmf, ue_mlp, ie_mlp,
           W0, b0, W1, b1, W2, b2, Wp, bp):
    info = plsc.get_sparse_core_info()
    gather = _build_sc_gather(info.num_cores, info.num_subcores)

    ugp_t, ump_t = _repack_tc(ue_gmf.T, ue_mlp.T)
    igp_t, imp_t = _repack_tc(ie_gmf.T, ie_mlp.T)

    ui = user_indices.astype(jnp.int32)
    ii = item_indices.astype(jnp.int32)
    urow = (ui // BN) * G4 + ui % G4
    irow = (ii // BN) * G4 + ii % G4
    ju = ((ui // G4) % 4).reshape(B, 1)
    ji = ((ii // G4) % 4).reshape(B, 1)

    ugr, igr, umr, imr = gather(
        urow.reshape(B // GCH, GCH), irow.reshape(B // GCH, GCH),
        ugp_t, igp_t, ump_t, imp_t)

    h2 = W2.shape[1]
    out = _dense_tc(
        ugr, igr, umr, imr, ju, ji,
        W0[:D], W0[D:], b0.reshape(1, -1),
        W1, b1.reshape(1, -1),
        W2, b2.reshape(1, -1),
        Wp[:D].reshape(1, D), Wp[D:].reshape(1, h2), bp.reshape(1, 1),
    )
    return out.reshape(-1)


# speedup vs baseline: 2.4874x; 1.5867x over previous
import functools

import jax
import jax.numpy as jnp
from jax import lax
from jax.experimental import pallas as pl
from jax.experimental.pallas import tpu as pltpu
from jax.experimental.pallas import tpu_sc as plsc

B = 16384
D = 32
BN = 8192
G4 = BN // 4
NV = 1000000
NBLK = (NV + BN - 1) // BN
N4 = NBLK * G4
GCH = 128


def _repack_body(ta_ref, tb_ref, out_ref):
    i = pl.program_id(0)
    cols = jax.lax.broadcasted_iota(jnp.int32, (D, BN), 1) + i * BN
    mask = cols < NV
    xa = jnp.where(mask, ta_ref[...], 0.0)
    xb = jnp.where(mask, tb_ref[...], 0.0)
    lanes = jax.lax.broadcasted_iota(jnp.int32, (D, 128), 1)
    chans = jax.lax.broadcasted_iota(jnp.int32, (D, 128), 0)
    acc_a = jnp.zeros((G4, 128), jnp.float32)
    acc_b = jnp.zeros((G4, 128), jnp.float32)
    for j in range(4):
        ej = (lanes == j * D + chans).astype(jnp.float32)
        acc_a = acc_a + jax.lax.dot_general(
            xa[:, j * G4:(j + 1) * G4], ej, (((0,), (0,)), ((), ())),
            preferred_element_type=jnp.float32)
        acc_b = acc_b + jax.lax.dot_general(
            xb[:, j * G4:(j + 1) * G4], ej, (((0,), (0,)), ((), ())),
            preferred_element_type=jnp.float32)
    wa = jax.lax.bitcast_convert_type(acc_a, jnp.uint32)
    wb = jax.lax.bitcast_convert_type(acc_b, jnp.uint32)
    hi = jnp.uint32(0xFFFF0000)
    out_ref[...] = (wa & hi) | (wb >> 16)


def _repack_tc(ta_t, tb_t):
    return pl.pallas_call(
        _repack_body,
        grid=(NBLK,),
        in_specs=[
            pl.BlockSpec((D, BN), lambda i: (0, i)),
            pl.BlockSpec((D, BN), lambda i: (0, i)),
        ],
        out_specs=pl.BlockSpec((G4, 128), lambda i: (i, 0)),
        out_shape=jax.ShapeDtypeStruct((N4, 128), jnp.uint32),
    )(ta_t, tb_t)


def _build_sc_gather(nc, ns):
    nw = nc * ns
    bpw = B // nw
    nch = bpw // GCH
    mesh = plsc.VectorSubcoreMesh(core_axis_name="c", subcore_axis_name="s")

    @functools.partial(
        pl.kernel,
        mesh=mesh,
        compiler_params=pltpu.CompilerParams(use_tc_tiling_on_sc=False),
        out_type=[jax.ShapeDtypeStruct((B, 128), jnp.uint32)] * 2,
        scratch_types=[
            pltpu.VMEM((nch, GCH), jnp.int32),
            pltpu.VMEM((nch, GCH), jnp.int32),
            pltpu.VMEM((GCH, 128), jnp.uint32),
            pltpu.VMEM((GCH, 128), jnp.uint32),
            pltpu.SemaphoreType.DMA,
            pltpu.SemaphoreType.DMA,
        ],
    )
    def gather_kernel(uidx_hbm, iidx_hbm, up_hbm, ip_hbm, ou, oi,
                      uidx_v, iidx_v, rows_a, rows_b, sem_a, sem_b):
        wid = lax.axis_index("s") * nc + lax.axis_index("c")
        base = wid * bpw
        pltpu.sync_copy(uidx_hbm.at[pl.ds(wid * nch, nch)], uidx_v)
        pltpu.sync_copy(iidx_hbm.at[pl.ds(wid * nch, nch)], iidx_v)

        seq = []
        for tab, idxv, outh in ((up_hbm, uidx_v, ou), (ip_hbm, iidx_v, oi)):
            for cc in range(nch):
                seq.append((tab, idxv, outh, cc))

        bufs = (rows_a, rows_b)
        sems = (sem_a, sem_b)

        def start(s):
            tab, idxv, _, cc = seq[s]
            pltpu.async_copy(tab.at[idxv.at[cc]], bufs[s % 2], sems[s % 2])

        start(0)
        for s in range(len(seq)):
            tab, idxv, outh, cc = seq[s]
            p = s % 2
            if s + 1 < len(seq):
                start(s + 1)
            pltpu.make_async_copy(tab.at[idxv.at[cc]], bufs[p], sems[p]).wait()
            pltpu.sync_copy(bufs[p], outh.at[pl.ds(base + cc * GCH, GCH)])

    return gather_kernel


BM = 2048


def _dense_body(upw, ipw, ju, ji, w0u, w0i, b0, w1, b1, w2, b2,
                wpg, wpm, bp, out):
    uw = upw[...]
    iw = ipw[...]
    hi = jnp.uint32(0xFFFF0000)
    ugp = jax.lax.bitcast_convert_type(uw & hi, jnp.float32)
    ump = jax.lax.bitcast_convert_type(uw << 16, jnp.float32)
    igp = jax.lax.bitcast_convert_type(iw & hi, jnp.float32)
    imp = jax.lax.bitcast_convert_type(iw << 16, jnp.float32)
    sel_u = ju[...]
    sel_i = ji[...]
    ug = jnp.zeros((BM, D), jnp.float32)
    um = jnp.zeros((BM, D), jnp.float32)
    ig = jnp.zeros((BM, D), jnp.float32)
    im = jnp.zeros((BM, D), jnp.float32)
    for j in range(4):
        mu = (sel_u == j).astype(jnp.float32)
        mi = (sel_i == j).astype(jnp.float32)
        ug = ug + ugp[:, j * D:(j + 1) * D] * mu
        um = um + ump[:, j * D:(j + 1) * D] * mu
        ig = ig + igp[:, j * D:(j + 1) * D] * mi
        im = im + imp[:, j * D:(j + 1) * D] * mi
    h = jnp.dot(um, w0u[...], preferred_element_type=jnp.float32)
    h = h + jnp.dot(im, w0i[...], preferred_element_type=jnp.float32)
    h = jnp.maximum(h + b0[...], 0.0)
    h = jnp.maximum(jnp.dot(h, w1[...], preferred_element_type=jnp.float32) + b1[...], 0.0)
    h = jnp.maximum(jnp.dot(h, w2[...], preferred_element_type=jnp.float32) + b2[...], 0.0)
    g = ug * ig
    logit = (jnp.sum(g * wpg[...], axis=1, keepdims=True)
             + jnp.sum(h * wpm[...], axis=1, keepdims=True) + bp[...])
    out[...] = 1.0 / (1.0 + jnp.exp(-logit))


def _dense_tc(upw, ipw, ju, ji, w0u, w0i, b0, w1, b1, w2, b2, wpg, wpm, bp):
    row = lambda i: (i, 0)
    rep = lambda i: (0, 0)
    h0, h1, h2 = b0.shape[1], b1.shape[1], b2.shape[1]
    return pl.pallas_call(
        _dense_body,
        grid=(B // BM,),
        in_specs=[
            pl.BlockSpec((BM, 128), row),
            pl.BlockSpec((BM, 128), row),
            pl.BlockSpec((BM, 1), row),
            pl.BlockSpec((BM, 1), row),
            pl.BlockSpec((D, h0), rep),
            pl.BlockSpec((D, h0), rep),
            pl.BlockSpec((1, h0), rep),
            pl.BlockSpec((h0, h1), rep),
            pl.BlockSpec((1, h1), rep),
            pl.BlockSpec((h1, h2), rep),
            pl.BlockSpec((1, h2), rep),
            pl.BlockSpec((1, D), rep),
            pl.BlockSpec((1, h2), rep),
            pl.BlockSpec((1, 1), rep),
        ],
        out_specs=pl.BlockSpec((BM, 1), row),
        out_shape=jax.ShapeDtypeStruct((B, 1), jnp.float32),
    )(upw, ipw, ju, ji, w0u, w0i, b0, w1, b1, w2, b2, wpg, wpm, bp)


def kernel(user_indices, item_indices, ue_gmf, ie_gmf, ue_mlp, ie_mlp,
           W0, b0, W1, b1, W2, b2, Wp, bp):
    info = plsc.get_sparse_core_info()
    gather = _build_sc_gather(info.num_cores, info.num_subcores)

    up = _repack_tc(ue_gmf.T, ue_mlp.T)
    ip = _repack_tc(ie_gmf.T, ie_mlp.T)

    ui = user_indices.astype(jnp.int32)
    ii = item_indices.astype(jnp.int32)
    urow = (ui // BN) * G4 + ui % G4
    irow = (ii // BN) * G4 + ii % G4
    ju = ((ui // G4) % 4).reshape(B, 1)
    ji = ((ii // G4) % 4).reshape(B, 1)

    upr, ipr = gather(
        urow.reshape(B // GCH, GCH), irow.reshape(B // GCH, GCH), up, ip)

    h2 = W2.shape[1]
    out = _dense_tc(
        upr, ipr, ju, ji,
        W0[:D], W0[D:], b0.reshape(1, -1),
        W1, b1.reshape(1, -1),
        W2, b2.reshape(1, -1),
        Wp[:D].reshape(1, D), Wp[D:].reshape(1, h2), bp.reshape(1, 1),
    )
    return out.reshape(-1)


# speedup vs baseline: 3.2954x; 1.3249x over previous
import functools

import jax
import jax.numpy as jnp
from jax import lax
from jax.experimental import pallas as pl
from jax.experimental.pallas import tpu as pltpu
from jax.experimental.pallas import tpu_sc as plsc

B = 16384
D = 32
BN = 8192
G4 = BN // 4
NV = 1000000
NBLK = (NV + BN - 1) // BN
N4 = NBLK * G4
GCH = 128


def _repack_body(ta_ref, tb_ref, out_ref):
    xa = ta_ref[...].astype(jnp.bfloat16)
    xb = tb_ref[...].astype(jnp.bfloat16)
    lanes = jax.lax.broadcasted_iota(jnp.int32, (D, 128), 1)
    chans = jax.lax.broadcasted_iota(jnp.int32, (D, 128), 0)
    acc_a = jnp.zeros((G4, 128), jnp.float32)
    acc_b = jnp.zeros((G4, 128), jnp.float32)
    for j in range(4):
        ej = (lanes == j * D + chans).astype(jnp.bfloat16)
        acc_a = acc_a + jax.lax.dot_general(
            xa[:, j * G4:(j + 1) * G4], ej, (((0,), (0,)), ((), ())),
            preferred_element_type=jnp.float32)
        acc_b = acc_b + jax.lax.dot_general(
            xb[:, j * G4:(j + 1) * G4], ej, (((0,), (0,)), ((), ())),
            preferred_element_type=jnp.float32)
    wa = jax.lax.bitcast_convert_type(acc_a, jnp.uint32)
    wb = jax.lax.bitcast_convert_type(acc_b, jnp.uint32)
    hi = jnp.uint32(0xFFFF0000)
    out_ref[...] = (wa & hi) | (wb >> 16)


def _repack_tc(ta_t, tb_t):
    return pl.pallas_call(
        _repack_body,
        grid=(NBLK,),
        in_specs=[
            pl.BlockSpec((D, BN), lambda i: (0, i)),
            pl.BlockSpec((D, BN), lambda i: (0, i)),
        ],
        out_specs=pl.BlockSpec((G4, 128), lambda i: (i, 0)),
        out_shape=jax.ShapeDtypeStruct((N4, 128), jnp.uint32),
    )(ta_t, tb_t)


def _build_sc_gather(nc, ns):
    nw = nc * ns
    bpw = B // nw
    nch = bpw // GCH
    mesh = plsc.VectorSubcoreMesh(core_axis_name="c", subcore_axis_name="s")

    @functools.partial(
        pl.kernel,
        mesh=mesh,
        compiler_params=pltpu.CompilerParams(use_tc_tiling_on_sc=False),
        out_type=[jax.ShapeDtypeStruct((B, 128), jnp.uint32)] * 2,
        scratch_types=[
            pltpu.VMEM((nch, GCH), jnp.int32),
            pltpu.VMEM((nch, GCH), jnp.int32),
            pltpu.VMEM((GCH, 128), jnp.uint32),
            pltpu.VMEM((GCH, 128), jnp.uint32),
            pltpu.SemaphoreType.DMA,
            pltpu.SemaphoreType.DMA,
        ],
    )
    def gather_kernel(uidx_hbm, iidx_hbm, up_hbm, ip_hbm, ou, oi,
                      uidx_v, iidx_v, rows_a, rows_b, sem_a, sem_b):
        wid = lax.axis_index("s") * nc + lax.axis_index("c")
        base = wid * bpw
        pltpu.sync_copy(uidx_hbm.at[pl.ds(wid * nch, nch)], uidx_v)
        pltpu.sync_copy(iidx_hbm.at[pl.ds(wid * nch, nch)], iidx_v)

        seq = []
        for tab, idxv, outh in ((up_hbm, uidx_v, ou), (ip_hbm, iidx_v, oi)):
            for cc in range(nch):
                seq.append((tab, idxv, outh, cc))

        bufs = (rows_a, rows_b)
        sems = (sem_a, sem_b)

        def start(s):
            tab, idxv, _, cc = seq[s]
            pltpu.async_copy(tab.at[idxv.at[cc]], bufs[s % 2], sems[s % 2])

        start(0)
        for s in range(len(seq)):
            tab, idxv, outh, cc = seq[s]
            p = s % 2
            if s + 1 < len(seq):
                start(s + 1)
            pltpu.make_async_copy(tab.at[idxv.at[cc]], bufs[p], sems[p]).wait()
            pltpu.sync_copy(bufs[p], outh.at[pl.ds(base + cc * GCH, GCH)])

    return gather_kernel


BM = 2048


def _dense_body(upw, ipw, ju, ji, w0u, w0i, b0, w1, b1, w2, b2,
                wpg, wpm, bp, out):
    uw = upw[...]
    iw = ipw[...]
    hi = jnp.uint32(0xFFFF0000)
    ugp = jax.lax.bitcast_convert_type(uw & hi, jnp.float32)
    ump = jax.lax.bitcast_convert_type(uw << 16, jnp.float32)
    igp = jax.lax.bitcast_convert_type(iw & hi, jnp.float32)
    imp = jax.lax.bitcast_convert_type(iw << 16, jnp.float32)
    sel_u = ju[...]
    sel_i = ji[...]
    ug = jnp.zeros((BM, D), jnp.float32)
    um = jnp.zeros((BM, D), jnp.float32)
    ig = jnp.zeros((BM, D), jnp.float32)
    im = jnp.zeros((BM, D), jnp.float32)
    for j in range(4):
        mu = sel_u == j
        mi = sel_i == j
        ug = ug + jnp.where(mu, ugp[:, j * D:(j + 1) * D], 0.0)
        um = um + jnp.where(mu, ump[:, j * D:(j + 1) * D], 0.0)
        ig = ig + jnp.where(mi, igp[:, j * D:(j + 1) * D], 0.0)
        im = im + jnp.where(mi, imp[:, j * D:(j + 1) * D], 0.0)
    h = jnp.dot(um, w0u[...], preferred_element_type=jnp.float32)
    h = h + jnp.dot(im, w0i[...], preferred_element_type=jnp.float32)
    h = jnp.maximum(h + b0[...], 0.0)
    h = jnp.maximum(jnp.dot(h, w1[...], preferred_element_type=jnp.float32) + b1[...], 0.0)
    h = jnp.maximum(jnp.dot(h, w2[...], preferred_element_type=jnp.float32) + b2[...], 0.0)
    g = ug * ig
    logit = (jnp.sum(g * wpg[...], axis=1, keepdims=True)
             + jnp.sum(h * wpm[...], axis=1, keepdims=True) + bp[...])
    out[...] = 1.0 / (1.0 + jnp.exp(-logit))


def _dense_tc(upw, ipw, ju, ji, w0u, w0i, b0, w1, b1, w2, b2, wpg, wpm, bp):
    row = lambda i: (i, 0)
    rep = lambda i: (0, 0)
    h0, h1, h2 = b0.shape[1], b1.shape[1], b2.shape[1]
    return pl.pallas_call(
        _dense_body,
        grid=(B // BM,),
        in_specs=[
            pl.BlockSpec((BM, 128), row),
            pl.BlockSpec((BM, 128), row),
            pl.BlockSpec((BM, 1), row),
            pl.BlockSpec((BM, 1), row),
            pl.BlockSpec((D, h0), rep),
            pl.BlockSpec((D, h0), rep),
            pl.BlockSpec((1, h0), rep),
            pl.BlockSpec((h0, h1), rep),
            pl.BlockSpec((1, h1), rep),
            pl.BlockSpec((h1, h2), rep),
            pl.BlockSpec((1, h2), rep),
            pl.BlockSpec((1, D), rep),
            pl.BlockSpec((1, h2), rep),
            pl.BlockSpec((1, 1), rep),
        ],
        out_specs=pl.BlockSpec((BM, 1), row),
        out_shape=jax.ShapeDtypeStruct((B, 1), jnp.float32),
    )(upw, ipw, ju, ji, w0u, w0i, b0, w1, b1, w2, b2, wpg, wpm, bp)


def kernel(user_indices, item_indices, ue_gmf, ie_gmf, ue_mlp, ie_mlp,
           W0, b0, W1, b1, W2, b2, Wp, bp):
    info = plsc.get_sparse_core_info()
    gather = _build_sc_gather(info.num_cores, info.num_subcores)

    up = _repack_tc(ue_gmf.T, ue_mlp.T)
    ip = _repack_tc(ie_gmf.T, ie_mlp.T)

    ui = user_indices.astype(jnp.int32)
    ii = item_indices.astype(jnp.int32)
    urow = (ui // BN) * G4 + ui % G4
    irow = (ii // BN) * G4 + ii % G4
    ju = ((ui // G4) % 4).reshape(B, 1)
    ji = ((ii // G4) % 4).reshape(B, 1)

    upr, ipr = gather(
        urow.reshape(B // GCH, GCH), irow.reshape(B // GCH, GCH), up, ip)

    h2 = W2.shape[1]
    out = _dense_tc(
        upr, ipr, ju, ji,
        W0[:D], W0[D:], b0.reshape(1, -1),
        W1, b1.reshape(1, -1),
        W2, b2.reshape(1, -1),
        Wp[:D].reshape(1, D), Wp[D:].reshape(1, h2), bp.reshape(1, 1),
    )
    return out.reshape(-1)


# speedup vs baseline: 3.8678x; 1.1737x over previous
import functools

import jax
import jax.numpy as jnp
from jax import lax
from jax.experimental import pallas as pl
from jax.experimental.pallas import tpu as pltpu
from jax.experimental.pallas import tpu_sc as plsc

B = 16384
D = 32
BN = 16384
G4 = BN // 4
NV = 1000000
NBLK = (NV + BN - 1) // BN
N4 = NBLK * G4
GCH = 128


def _repack_body(ta_ref, tb_ref, out_ref):
    xa = ta_ref[...].astype(jnp.bfloat16)
    xb = tb_ref[...].astype(jnp.bfloat16)
    lanes = jax.lax.broadcasted_iota(jnp.int32, (D, 128), 1)
    chans = jax.lax.broadcasted_iota(jnp.int32, (D, 128), 0)
    acc_a = jnp.zeros((G4, 128), jnp.float32)
    acc_b = jnp.zeros((G4, 128), jnp.float32)
    for j in range(4):
        ej = (lanes == j * D + chans).astype(jnp.bfloat16)
        acc_a = acc_a + jax.lax.dot_general(
            xa[:, j * G4:(j + 1) * G4], ej, (((0,), (0,)), ((), ())),
            preferred_element_type=jnp.float32)
        acc_b = acc_b + jax.lax.dot_general(
            xb[:, j * G4:(j + 1) * G4], ej, (((0,), (0,)), ((), ())),
            preferred_element_type=jnp.float32)
    wa = jax.lax.bitcast_convert_type(acc_a, jnp.uint32)
    wb = jax.lax.bitcast_convert_type(acc_b, jnp.uint32)
    hi = jnp.uint32(0xFFFF0000)
    out_ref[...] = (wa & hi) | (wb >> 16)


def _repack_tc(ta_t, tb_t):
    return pl.pallas_call(
        _repack_body,
        grid=(NBLK,),
        in_specs=[
            pl.BlockSpec((D, BN), lambda i: (0, i)),
            pl.BlockSpec((D, BN), lambda i: (0, i)),
        ],
        out_specs=pl.BlockSpec((G4, 128), lambda i: (i, 0)),
        out_shape=jax.ShapeDtypeStruct((N4, 128), jnp.uint32),
    )(ta_t, tb_t)


def _build_sc_gather(nc, ns):
    nw = nc * ns
    bpw = B // nw
    nch = bpw // GCH
    mesh = plsc.VectorSubcoreMesh(core_axis_name="c", subcore_axis_name="s")

    @functools.partial(
        pl.kernel,
        mesh=mesh,
        compiler_params=pltpu.CompilerParams(use_tc_tiling_on_sc=False),
        out_type=[jax.ShapeDtypeStruct((B, 128), jnp.uint32)] * 2,
        scratch_types=[
            pltpu.VMEM((nch, GCH), jnp.int32),
            pltpu.VMEM((nch, GCH), jnp.int32),
            pltpu.VMEM((GCH, 128), jnp.uint32),
            pltpu.VMEM((GCH, 128), jnp.uint32),
            pltpu.SemaphoreType.DMA,
            pltpu.SemaphoreType.DMA,
        ],
    )
    def gather_kernel(uidx_hbm, iidx_hbm, up_hbm, ip_hbm, ou, oi,
                      uidx_v, iidx_v, rows_a, rows_b, sem_a, sem_b):
        wid = lax.axis_index("s") * nc + lax.axis_index("c")
        base = wid * bpw
        pltpu.sync_copy(uidx_hbm.at[pl.ds(wid * nch, nch)], uidx_v)
        pltpu.sync_copy(iidx_hbm.at[pl.ds(wid * nch, nch)], iidx_v)

        seq = []
        for tab, idxv, outh in ((up_hbm, uidx_v, ou), (ip_hbm, iidx_v, oi)):
            for cc in range(nch):
                seq.append((tab, idxv, outh, cc))

        bufs = (rows_a, rows_b)
        sems = (sem_a, sem_b)

        def start(s):
            tab, idxv, _, cc = seq[s]
            pltpu.async_copy(tab.at[idxv.at[cc]], bufs[s % 2], sems[s % 2])

        start(0)
        for s in range(len(seq)):
            tab, idxv, outh, cc = seq[s]
            p = s % 2
            if s + 1 < len(seq):
                start(s + 1)
            pltpu.make_async_copy(tab.at[idxv.at[cc]], bufs[p], sems[p]).wait()
            pltpu.sync_copy(bufs[p], outh.at[pl.ds(base + cc * GCH, GCH)])

    return gather_kernel


BM = 2048


def _dense_body(upw, ipw, ju, ji, w0u, w0i, b0, w1, b1, w2, b2,
                wpg, wpm, bp, out):
    uw = upw[...]
    iw = ipw[...]
    hi = jnp.uint32(0xFFFF0000)
    ugp = jax.lax.bitcast_convert_type(uw & hi, jnp.float32)
    ump = jax.lax.bitcast_convert_type(uw << 16, jnp.float32)
    igp = jax.lax.bitcast_convert_type(iw & hi, jnp.float32)
    imp = jax.lax.bitcast_convert_type(iw << 16, jnp.float32)
    sel_u = ju[...]
    sel_i = ji[...]
    ug = jnp.zeros((BM, D), jnp.float32)
    um = jnp.zeros((BM, D), jnp.float32)
    ig = jnp.zeros((BM, D), jnp.float32)
    im = jnp.zeros((BM, D), jnp.float32)
    for j in range(4):
        mu = sel_u == j
        mi = sel_i == j
        ug = ug + jnp.where(mu, ugp[:, j * D:(j + 1) * D], 0.0)
        um = um + jnp.where(mu, ump[:, j * D:(j + 1) * D], 0.0)
        ig = ig + jnp.where(mi, igp[:, j * D:(j + 1) * D], 0.0)
        im = im + jnp.where(mi, imp[:, j * D:(j + 1) * D], 0.0)
    h = jnp.dot(um, w0u[...], preferred_element_type=jnp.float32)
    h = h + jnp.dot(im, w0i[...], preferred_element_type=jnp.float32)
    h = jnp.maximum(h + b0[...], 0.0)
    h = jnp.maximum(jnp.dot(h, w1[...], preferred_element_type=jnp.float32) + b1[...], 0.0)
    h = jnp.maximum(jnp.dot(h, w2[...], preferred_element_type=jnp.float32) + b2[...], 0.0)
    g = ug * ig
    logit = (jnp.sum(g * wpg[...], axis=1, keepdims=True)
             + jnp.sum(h * wpm[...], axis=1, keepdims=True) + bp[...])
    out[...] = 1.0 / (1.0 + jnp.exp(-logit))


def _dense_tc(upw, ipw, ju, ji, w0u, w0i, b0, w1, b1, w2, b2, wpg, wpm, bp):
    row = lambda i: (i, 0)
    rep = lambda i: (0, 0)
    h0, h1, h2 = b0.shape[1], b1.shape[1], b2.shape[1]
    return pl.pallas_call(
        _dense_body,
        grid=(B // BM,),
        in_specs=[
            pl.BlockSpec((BM, 128), row),
            pl.BlockSpec((BM, 128), row),
            pl.BlockSpec((BM, 1), row),
            pl.BlockSpec((BM, 1), row),
            pl.BlockSpec((D, h0), rep),
            pl.BlockSpec((D, h0), rep),
            pl.BlockSpec((1, h0), rep),
            pl.BlockSpec((h0, h1), rep),
            pl.BlockSpec((1, h1), rep),
            pl.BlockSpec((h1, h2), rep),
            pl.BlockSpec((1, h2), rep),
            pl.BlockSpec((1, D), rep),
            pl.BlockSpec((1, h2), rep),
            pl.BlockSpec((1, 1), rep),
        ],
        out_specs=pl.BlockSpec((BM, 1), row),
        out_shape=jax.ShapeDtypeStruct((B, 1), jnp.float32),
    )(upw, ipw, ju, ji, w0u, w0i, b0, w1, b1, w2, b2, wpg, wpm, bp)


def kernel(user_indices, item_indices, ue_gmf, ie_gmf, ue_mlp, ie_mlp,
           W0, b0, W1, b1, W2, b2, Wp, bp):
    info = plsc.get_sparse_core_info()
    gather = _build_sc_gather(info.num_cores, info.num_subcores)

    up = _repack_tc(ue_gmf.T, ue_mlp.T)
    ip = _repack_tc(ie_gmf.T, ie_mlp.T)

    ui = user_indices.astype(jnp.int32)
    ii = item_indices.astype(jnp.int32)
    urow = (ui // BN) * G4 + ui % G4
    irow = (ii // BN) * G4 + ii % G4
    ju = ((ui // G4) % 4).reshape(B, 1)
    ji = ((ii // G4) % 4).reshape(B, 1)

    upr, ipr = gather(
        urow.reshape(B // GCH, GCH), irow.reshape(B // GCH, GCH), up, ip)

    h2 = W2.shape[1]
    out = _dense_tc(
        upr, ipr, ju, ji,
        W0[:D], W0[D:], b0.reshape(1, -1),
        W1, b1.reshape(1, -1),
        W2, b2.reshape(1, -1),
        Wp[:D].reshape(1, D), Wp[D:].reshape(1, h2), bp.reshape(1, 1),
    )
    return out.reshape(-1)


# speedup vs baseline: 4.2522x; 1.0994x over previous
import functools

import jax
import jax.numpy as jnp
from jax import lax
from jax.experimental import pallas as pl
from jax.experimental.pallas import tpu as pltpu
from jax.experimental.pallas import tpu_sc as plsc

B = 16384
D = 32
BN = 32768
G4 = BN // 4
NV = 1000000
NBLK = (NV + BN - 1) // BN
N4 = NBLK * G4
GCH = 128


def _repack_body(ta_ref, tb_ref, out_ref):
    xa = ta_ref[...].astype(jnp.bfloat16)
    xb = tb_ref[...].astype(jnp.bfloat16)
    lanes = jax.lax.broadcasted_iota(jnp.int32, (D, 128), 1)
    chans = jax.lax.broadcasted_iota(jnp.int32, (D, 128), 0)
    acc_a = jnp.zeros((G4, 128), jnp.float32)
    acc_b = jnp.zeros((G4, 128), jnp.float32)
    for j in range(4):
        ej = (lanes == j * D + chans).astype(jnp.bfloat16)
        acc_a = acc_a + jax.lax.dot_general(
            xa[:, j * G4:(j + 1) * G4], ej, (((0,), (0,)), ((), ())),
            preferred_element_type=jnp.float32)
        acc_b = acc_b + jax.lax.dot_general(
            xb[:, j * G4:(j + 1) * G4], ej, (((0,), (0,)), ((), ())),
            preferred_element_type=jnp.float32)
    wa = jax.lax.bitcast_convert_type(acc_a, jnp.uint32)
    wb = jax.lax.bitcast_convert_type(acc_b, jnp.uint32)
    hi = jnp.uint32(0xFFFF0000)
    out_ref[...] = (wa & hi) | (wb >> 16)


def _repack_tc(ta_t, tb_t):
    return pl.pallas_call(
        _repack_body,
        grid=(NBLK,),
        in_specs=[
            pl.BlockSpec((D, BN), lambda i: (0, i)),
            pl.BlockSpec((D, BN), lambda i: (0, i)),
        ],
        out_specs=pl.BlockSpec((G4, 128), lambda i: (i, 0)),
        out_shape=jax.ShapeDtypeStruct((N4, 128), jnp.uint32),
    )(ta_t, tb_t)


def _build_sc_gather(nc, ns):
    nw = nc * ns
    bpw = B // nw
    nch = bpw // GCH
    mesh = plsc.VectorSubcoreMesh(core_axis_name="c", subcore_axis_name="s")

    @functools.partial(
        pl.kernel,
        mesh=mesh,
        compiler_params=pltpu.CompilerParams(use_tc_tiling_on_sc=False),
        out_type=[jax.ShapeDtypeStruct((B, 128), jnp.uint32)] * 2,
        scratch_types=[
            pltpu.VMEM((nch, GCH), jnp.int32),
            pltpu.VMEM((nch, GCH), jnp.int32),
            pltpu.VMEM((GCH, 128), jnp.uint32),
            pltpu.VMEM((GCH, 128), jnp.uint32),
            pltpu.SemaphoreType.DMA,
            pltpu.SemaphoreType.DMA,
        ],
    )
    def gather_kernel(uidx_hbm, iidx_hbm, up_hbm, ip_hbm, ou, oi,
                      uidx_v, iidx_v, rows_a, rows_b, sem_a, sem_b):
        wid = lax.axis_index("s") * nc + lax.axis_index("c")
        base = wid * bpw
        pltpu.sync_copy(uidx_hbm.at[pl.ds(wid * nch, nch)], uidx_v)
        pltpu.sync_copy(iidx_hbm.at[pl.ds(wid * nch, nch)], iidx_v)

        seq = []
        for tab, idxv, outh in ((up_hbm, uidx_v, ou), (ip_hbm, iidx_v, oi)):
            for cc in range(nch):
                seq.append((tab, idxv, outh, cc))

        bufs = (rows_a, rows_b)
        sems = (sem_a, sem_b)

        def start(s):
            tab, idxv, _, cc = seq[s]
            pltpu.async_copy(tab.at[idxv.at[cc]], bufs[s % 2], sems[s % 2])

        start(0)
        for s in range(len(seq)):
            tab, idxv, outh, cc = seq[s]
            p = s % 2
            if s + 1 < len(seq):
                start(s + 1)
            pltpu.make_async_copy(tab.at[idxv.at[cc]], bufs[p], sems[p]).wait()
            pltpu.sync_copy(bufs[p], outh.at[pl.ds(base + cc * GCH, GCH)])

    return gather_kernel


BM = 2048


def _dense_body(upw, ipw, ju, ji, w0u, w0i, b0, w1, b1, w2, b2,
                wpg, wpm, bp, out):
    uw = upw[...]
    iw = ipw[...]
    hi = jnp.uint32(0xFFFF0000)
    ugp = jax.lax.bitcast_convert_type(uw & hi, jnp.float32)
    ump = jax.lax.bitcast_convert_type(uw << 16, jnp.float32)
    igp = jax.lax.bitcast_convert_type(iw & hi, jnp.float32)
    imp = jax.lax.bitcast_convert_type(iw << 16, jnp.float32)
    sel_u = ju[...]
    sel_i = ji[...]
    ug = jnp.zeros((BM, D), jnp.float32)
    um = jnp.zeros((BM, D), jnp.float32)
    ig = jnp.zeros((BM, D), jnp.float32)
    im = jnp.zeros((BM, D), jnp.float32)
    for j in range(4):
        mu = sel_u == j
        mi = sel_i == j
        ug = ug + jnp.where(mu, ugp[:, j * D:(j + 1) * D], 0.0)
        um = um + jnp.where(mu, ump[:, j * D:(j + 1) * D], 0.0)
        ig = ig + jnp.where(mi, igp[:, j * D:(j + 1) * D], 0.0)
        im = im + jnp.where(mi, imp[:, j * D:(j + 1) * D], 0.0)
    h = jnp.dot(um, w0u[...], preferred_element_type=jnp.float32)
    h = h + jnp.dot(im, w0i[...], preferred_element_type=jnp.float32)
    h = jnp.maximum(h + b0[...], 0.0)
    h = jnp.maximum(jnp.dot(h, w1[...], preferred_element_type=jnp.float32) + b1[...], 0.0)
    h = jnp.maximum(jnp.dot(h, w2[...], preferred_element_type=jnp.float32) + b2[...], 0.0)
    g = ug * ig
    logit = (jnp.sum(g * wpg[...], axis=1, keepdims=True)
             + jnp.sum(h * wpm[...], axis=1, keepdims=True) + bp[...])
    out[...] = 1.0 / (1.0 + jnp.exp(-logit))


def _dense_tc(upw, ipw, ju, ji, w0u, w0i, b0, w1, b1, w2, b2, wpg, wpm, bp):
    row = lambda i: (i, 0)
    rep = lambda i: (0, 0)
    h0, h1, h2 = b0.shape[1], b1.shape[1], b2.shape[1]
    return pl.pallas_call(
        _dense_body,
        grid=(B // BM,),
        in_specs=[
            pl.BlockSpec((BM, 128), row),
            pl.BlockSpec((BM, 128), row),
            pl.BlockSpec((BM, 1), row),
            pl.BlockSpec((BM, 1), row),
            pl.BlockSpec((D, h0), rep),
            pl.BlockSpec((D, h0), rep),
            pl.BlockSpec((1, h0), rep),
            pl.BlockSpec((h0, h1), rep),
            pl.BlockSpec((1, h1), rep),
            pl.BlockSpec((h1, h2), rep),
            pl.BlockSpec((1, h2), rep),
            pl.BlockSpec((1, D), rep),
            pl.BlockSpec((1, h2), rep),
            pl.BlockSpec((1, 1), rep),
        ],
        out_specs=pl.BlockSpec((BM, 1), row),
        out_shape=jax.ShapeDtypeStruct((B, 1), jnp.float32),
    )(upw, ipw, ju, ji, w0u, w0i, b0, w1, b1, w2, b2, wpg, wpm, bp)


def kernel(user_indices, item_indices, ue_gmf, ie_gmf, ue_mlp, ie_mlp,
           W0, b0, W1, b1, W2, b2, Wp, bp):
    info = plsc.get_sparse_core_info()
    gather = _build_sc_gather(info.num_cores, info.num_subcores)

    up = _repack_tc(ue_gmf.T, ue_mlp.T)
    ip = _repack_tc(ie_gmf.T, ie_mlp.T)

    ui = user_indices.astype(jnp.int32)
    ii = item_indices.astype(jnp.int32)
    urow = (ui // BN) * G4 + ui % G4
    irow = (ii // BN) * G4 + ii % G4
    ju = ((ui // G4) % 4).reshape(B, 1)
    ji = ((ii // G4) % 4).reshape(B, 1)

    upr, ipr = gather(
        urow.reshape(B // GCH, GCH), irow.reshape(B // GCH, GCH), up, ip)

    h2 = W2.shape[1]
    out = _dense_tc(
        upr, ipr, ju, ji,
        W0[:D], W0[D:], b0.reshape(1, -1),
        W1, b1.reshape(1, -1),
        W2, b2.reshape(1, -1),
        Wp[:D].reshape(1, D), Wp[D:].reshape(1, h2), bp.reshape(1, 1),
    )
    return out.reshape(-1)


# speedup vs baseline: 4.3093x; 1.0134x over previous
import functools

import jax
import jax.numpy as jnp
from jax import lax
from jax.experimental import pallas as pl
from jax.experimental.pallas import tpu as pltpu
from jax.experimental.pallas import tpu_sc as plsc

B = 16384
D = 32
BN = 49152
G4 = BN // 4
NV = 1000000
NBLK = (NV + BN - 1) // BN
N4 = NBLK * G4
GCH = 128


def _repack_body(ta_ref, tb_ref, out_ref):
    xa = ta_ref[...].astype(jnp.bfloat16)
    xb = tb_ref[...].astype(jnp.bfloat16)
    lanes = jax.lax.broadcasted_iota(jnp.int32, (D, 128), 1)
    chans = jax.lax.broadcasted_iota(jnp.int32, (D, 128), 0)
    acc_a = jnp.zeros((G4, 128), jnp.float32)
    acc_b = jnp.zeros((G4, 128), jnp.float32)
    for j in range(4):
        ej = (lanes == j * D + chans).astype(jnp.bfloat16)
        acc_a = acc_a + jax.lax.dot_general(
            xa[:, j * G4:(j + 1) * G4], ej, (((0,), (0,)), ((), ())),
            preferred_element_type=jnp.float32)
        acc_b = acc_b + jax.lax.dot_general(
            xb[:, j * G4:(j + 1) * G4], ej, (((0,), (0,)), ((), ())),
            preferred_element_type=jnp.float32)
    wa = jax.lax.bitcast_convert_type(acc_a, jnp.uint32)
    wb = jax.lax.bitcast_convert_type(acc_b, jnp.uint32)
    hi = jnp.uint32(0xFFFF0000)
    out_ref[...] = (wa & hi) | (wb >> 16)


def _repack_tc(ta_t, tb_t):
    return pl.pallas_call(
        _repack_body,
        grid=(NBLK,),
        in_specs=[
            pl.BlockSpec((D, BN), lambda i: (0, i)),
            pl.BlockSpec((D, BN), lambda i: (0, i)),
        ],
        out_specs=pl.BlockSpec((G4, 128), lambda i: (i, 0)),
        out_shape=jax.ShapeDtypeStruct((N4, 128), jnp.uint32),
    )(ta_t, tb_t)


def _build_sc_gather(nc, ns):
    nw = nc * ns
    bpw = B // nw
    nch = bpw // GCH
    mesh = plsc.VectorSubcoreMesh(core_axis_name="c", subcore_axis_name="s")

    @functools.partial(
        pl.kernel,
        mesh=mesh,
        compiler_params=pltpu.CompilerParams(use_tc_tiling_on_sc=False),
        out_type=[jax.ShapeDtypeStruct((B, 128), jnp.uint32)] * 2,
        scratch_types=[
            pltpu.VMEM((nch, GCH), jnp.int32),
            pltpu.VMEM((nch, GCH), jnp.int32),
            pltpu.VMEM((GCH, 128), jnp.uint32),
            pltpu.VMEM((GCH, 128), jnp.uint32),
            pltpu.SemaphoreType.DMA,
            pltpu.SemaphoreType.DMA,
        ],
    )
    def gather_kernel(uidx_hbm, iidx_hbm, up_hbm, ip_hbm, ou, oi,
                      uidx_v, iidx_v, rows_a, rows_b, sem_a, sem_b):
        wid = lax.axis_index("s") * nc + lax.axis_index("c")
        base = wid * bpw
        pltpu.sync_copy(uidx_hbm.at[pl.ds(wid * nch, nch)], uidx_v)
        pltpu.sync_copy(iidx_hbm.at[pl.ds(wid * nch, nch)], iidx_v)

        seq = []
        for tab, idxv, outh in ((up_hbm, uidx_v, ou), (ip_hbm, iidx_v, oi)):
            for cc in range(nch):
                seq.append((tab, idxv, outh, cc))

        bufs = (rows_a, rows_b)
        sems = (sem_a, sem_b)

        def start(s):
            tab, idxv, _, cc = seq[s]
            pltpu.async_copy(tab.at[idxv.at[cc]], bufs[s % 2], sems[s % 2])

        start(0)
        for s in range(len(seq)):
            tab, idxv, outh, cc = seq[s]
            p = s % 2
            if s + 1 < len(seq):
                start(s + 1)
            pltpu.make_async_copy(tab.at[idxv.at[cc]], bufs[p], sems[p]).wait()
            pltpu.sync_copy(bufs[p], outh.at[pl.ds(base + cc * GCH, GCH)])

    return gather_kernel


BM = 2048


def _dense_body(upw, ipw, ju, ji, w0u, w0i, b0, w1, b1, w2, b2,
                wpg, wpm, bp, out):
    uw = upw[...]
    iw = ipw[...]
    hi = jnp.uint32(0xFFFF0000)
    ugp = jax.lax.bitcast_convert_type(uw & hi, jnp.float32)
    ump = jax.lax.bitcast_convert_type(uw << 16, jnp.float32)
    igp = jax.lax.bitcast_convert_type(iw & hi, jnp.float32)
    imp = jax.lax.bitcast_convert_type(iw << 16, jnp.float32)
    sel_u = ju[...]
    sel_i = ji[...]
    ug = jnp.zeros((BM, D), jnp.float32)
    um = jnp.zeros((BM, D), jnp.float32)
    ig = jnp.zeros((BM, D), jnp.float32)
    im = jnp.zeros((BM, D), jnp.float32)
    for j in range(4):
        mu = sel_u == j
        mi = sel_i == j
        ug = ug + jnp.where(mu, ugp[:, j * D:(j + 1) * D], 0.0)
        um = um + jnp.where(mu, ump[:, j * D:(j + 1) * D], 0.0)
        ig = ig + jnp.where(mi, igp[:, j * D:(j + 1) * D], 0.0)
        im = im + jnp.where(mi, imp[:, j * D:(j + 1) * D], 0.0)
    h = jnp.dot(um, w0u[...], preferred_element_type=jnp.float32)
    h = h + jnp.dot(im, w0i[...], preferred_element_type=jnp.float32)
    h = jnp.maximum(h + b0[...], 0.0)
    h = jnp.maximum(jnp.dot(h, w1[...], preferred_element_type=jnp.float32) + b1[...], 0.0)
    h = jnp.maximum(jnp.dot(h, w2[...], preferred_element_type=jnp.float32) + b2[...], 0.0)
    g = ug * ig
    logit = (jnp.sum(g * wpg[...], axis=1, keepdims=True)
             + jnp.sum(h * wpm[...], axis=1, keepdims=True) + bp[...])
    out[...] = 1.0 / (1.0 + jnp.exp(-logit))


def _dense_tc(upw, ipw, ju, ji, w0u, w0i, b0, w1, b1, w2, b2, wpg, wpm, bp):
    row = lambda i: (i, 0)
    rep = lambda i: (0, 0)
    h0, h1, h2 = b0.shape[1], b1.shape[1], b2.shape[1]
    return pl.pallas_call(
        _dense_body,
        grid=(B // BM,),
        in_specs=[
            pl.BlockSpec((BM, 128), row),
            pl.BlockSpec((BM, 128), row),
            pl.BlockSpec((BM, 1), row),
            pl.BlockSpec((BM, 1), row),
            pl.BlockSpec((D, h0), rep),
            pl.BlockSpec((D, h0), rep),
            pl.BlockSpec((1, h0), rep),
            pl.BlockSpec((h0, h1), rep),
            pl.BlockSpec((1, h1), rep),
            pl.BlockSpec((h1, h2), rep),
            pl.BlockSpec((1, h2), rep),
            pl.BlockSpec((1, D), rep),
            pl.BlockSpec((1, h2), rep),
            pl.BlockSpec((1, 1), rep),
        ],
        out_specs=pl.BlockSpec((BM, 1), row),
        out_shape=jax.ShapeDtypeStruct((B, 1), jnp.float32),
    )(upw, ipw, ju, ji, w0u, w0i, b0, w1, b1, w2, b2, wpg, wpm, bp)


def kernel(user_indices, item_indices, ue_gmf, ie_gmf, ue_mlp, ie_mlp,
           W0, b0, W1, b1, W2, b2, Wp, bp):
    info = plsc.get_sparse_core_info()
    gather = _build_sc_gather(info.num_cores, info.num_subcores)

    up = _repack_tc(ue_gmf.T, ue_mlp.T)
    ip = _repack_tc(ie_gmf.T, ie_mlp.T)

    ui = user_indices.astype(jnp.int32)
    ii = item_indices.astype(jnp.int32)
    urow = (ui // BN) * G4 + ui % G4
    irow = (ii // BN) * G4 + ii % G4
    ju = ((ui // G4) % 4).reshape(B, 1)
    ji = ((ii // G4) % 4).reshape(B, 1)

    upr, ipr = gather(
        urow.reshape(B // GCH, GCH), irow.reshape(B // GCH, GCH), up, ip)

    h2 = W2.shape[1]
    out = _dense_tc(
        upr, ipr, ju, ji,
        W0[:D], W0[D:], b0.reshape(1, -1),
        W1, b1.reshape(1, -1),
        W2, b2.reshape(1, -1),
        Wp[:D].reshape(1, D), Wp[D:].reshape(1, h2), bp.reshape(1, 1),
    )
    return out.reshape(-1)


# speedup vs baseline: 4.3366x; 1.0063x over previous
import functools

import jax
import jax.numpy as jnp
from jax import lax
from jax.experimental import pallas as pl
from jax.experimental.pallas import tpu as pltpu
from jax.experimental.pallas import tpu_sc as plsc

B = 16384
D = 32
BN = 49152
G4 = BN // 4
NV = 1000000
NBLK = (NV + BN - 1) // BN
N4 = NBLK * G4
GCH = 128


def _repack_body(ta_ref, tb_ref, out_ref):
    xa = ta_ref[...].astype(jnp.bfloat16)
    xb = tb_ref[...].astype(jnp.bfloat16)
    lanes = jax.lax.broadcasted_iota(jnp.int32, (D, 128), 1)
    chans = jax.lax.broadcasted_iota(jnp.int32, (D, 128), 0)
    acc_a = jnp.zeros((G4, 128), jnp.float32)
    acc_b = jnp.zeros((G4, 128), jnp.float32)
    for j in range(4):
        ej = (lanes == j * D + chans).astype(jnp.bfloat16)
        acc_a = acc_a + jax.lax.dot_general(
            xa[:, j * G4:(j + 1) * G4], ej, (((0,), (0,)), ((), ())),
            preferred_element_type=jnp.float32)
        acc_b = acc_b + jax.lax.dot_general(
            xb[:, j * G4:(j + 1) * G4], ej, (((0,), (0,)), ((), ())),
            preferred_element_type=jnp.float32)
    wa = jax.lax.bitcast_convert_type(acc_a, jnp.uint32)
    wb = jax.lax.bitcast_convert_type(acc_b, jnp.uint32)
    out_ref[...] = wa | (wb >> 16)


def _repack_tc(ta_t, tb_t):
    return pl.pallas_call(
        _repack_body,
        grid=(NBLK,),
        in_specs=[
            pl.BlockSpec((D, BN), lambda i: (0, i)),
            pl.BlockSpec((D, BN), lambda i: (0, i)),
        ],
        out_specs=pl.BlockSpec((G4, 128), lambda i: (i, 0)),
        out_shape=jax.ShapeDtypeStruct((N4, 128), jnp.uint32),
    )(ta_t, tb_t)


def _build_sc_gather(nc, ns):
    nw = nc * ns
    bpw = B // nw
    nch = bpw // GCH
    mesh = plsc.VectorSubcoreMesh(core_axis_name="c", subcore_axis_name="s")

    @functools.partial(
        pl.kernel,
        mesh=mesh,
        compiler_params=pltpu.CompilerParams(use_tc_tiling_on_sc=False),
        out_type=jax.ShapeDtypeStruct((B, 128), jnp.uint32),
        scratch_types=[
            pltpu.VMEM((nch, GCH), jnp.int32),
            pltpu.VMEM((GCH, 128), jnp.uint32),
            pltpu.VMEM((GCH, 128), jnp.uint32),
            pltpu.SemaphoreType.DMA,
            pltpu.SemaphoreType.DMA,
        ],
    )
    def gather_kernel(idx_hbm, tab_hbm, out_hbm,
                      idx_v, rows_a, rows_b, sem_a, sem_b):
        wid = lax.axis_index("s") * nc + lax.axis_index("c")
        base = wid * bpw
        pltpu.sync_copy(idx_hbm.at[pl.ds(wid * nch, nch)], idx_v)

        bufs = (rows_a, rows_b)
        sems = (sem_a, sem_b)

        def start(s):
            pltpu.async_copy(tab_hbm.at[idx_v.at[s]], bufs[s % 2], sems[s % 2])

        start(0)
        for s in range(nch):
            p = s % 2
            if s + 1 < nch:
                start(s + 1)
            pltpu.make_async_copy(
                tab_hbm.at[idx_v.at[s]], bufs[p], sems[p]).wait()
            pltpu.sync_copy(bufs[p], out_hbm.at[pl.ds(base + s * GCH, GCH)])

    return gather_kernel


BM = 2048


def _dense_body(upw, ipw, ju, ji, w0u, w0i, b0, w1, b1, w2, b2,
                wpg, wpm, bp, out):
    uw = upw[...]
    iw = ipw[...]
    hi = jnp.uint32(0xFFFF0000)
    ugp = jax.lax.bitcast_convert_type(uw & hi, jnp.float32)
    ump = jax.lax.bitcast_convert_type(uw << 16, jnp.float32)
    igp = jax.lax.bitcast_convert_type(iw & hi, jnp.float32)
    imp = jax.lax.bitcast_convert_type(iw << 16, jnp.float32)
    sel_u = ju[...]
    sel_i = ji[...]
    ug = jnp.zeros((BM, D), jnp.float32)
    um = jnp.zeros((BM, D), jnp.float32)
    ig = jnp.zeros((BM, D), jnp.float32)
    im = jnp.zeros((BM, D), jnp.float32)
    for j in range(4):
        mu = sel_u == j
        mi = sel_i == j
        ug = ug + jnp.where(mu, ugp[:, j * D:(j + 1) * D], 0.0)
        um = um + jnp.where(mu, ump[:, j * D:(j + 1) * D], 0.0)
        ig = ig + jnp.where(mi, igp[:, j * D:(j + 1) * D], 0.0)
        im = im + jnp.where(mi, imp[:, j * D:(j + 1) * D], 0.0)
    h = jnp.dot(um, w0u[...], preferred_element_type=jnp.float32)
    h = h + jnp.dot(im, w0i[...], preferred_element_type=jnp.float32)
    h = jnp.maximum(h + b0[...], 0.0)
    h = jnp.maximum(jnp.dot(h, w1[...], preferred_element_type=jnp.float32) + b1[...], 0.0)
    h = jnp.maximum(jnp.dot(h, w2[...], preferred_element_type=jnp.float32) + b2[...], 0.0)
    g = ug * ig
    logit = (jnp.sum(g * wpg[...], axis=1, keepdims=True)
             + jnp.sum(h * wpm[...], axis=1, keepdims=True) + bp[...])
    out[...] = 1.0 / (1.0 + jnp.exp(-logit))


def _dense_tc(upw, ipw, ju, ji, w0u, w0i, b0, w1, b1, w2, b2, wpg, wpm, bp):
    row = lambda i: (i, 0)
    rep = lambda i: (0, 0)
    h0, h1, h2 = b0.shape[1], b1.shape[1], b2.shape[1]
    return pl.pallas_call(
        _dense_body,
        grid=(B // BM,),
        in_specs=[
            pl.BlockSpec((BM, 128), row),
            pl.BlockSpec((BM, 128), row),
            pl.BlockSpec((BM, 1), row),
            pl.BlockSpec((BM, 1), row),
            pl.BlockSpec((D, h0), rep),
            pl.BlockSpec((D, h0), rep),
            pl.BlockSpec((1, h0), rep),
            pl.BlockSpec((h0, h1), rep),
            pl.BlockSpec((1, h1), rep),
            pl.BlockSpec((h1, h2), rep),
            pl.BlockSpec((1, h2), rep),
            pl.BlockSpec((1, D), rep),
            pl.BlockSpec((1, h2), rep),
            pl.BlockSpec((1, 1), rep),
        ],
        out_specs=pl.BlockSpec((BM, 1), row),
        out_shape=jax.ShapeDtypeStruct((B, 1), jnp.float32),
    )(upw, ipw, ju, ji, w0u, w0i, b0, w1, b1, w2, b2, wpg, wpm, bp)


def kernel(user_indices, item_indices, ue_gmf, ie_gmf, ue_mlp, ie_mlp,
           W0, b0, W1, b1, W2, b2, Wp, bp):
    info = plsc.get_sparse_core_info()
    gather = _build_sc_gather(info.num_cores, info.num_subcores)

    ui = user_indices.astype(jnp.int32)
    ii = item_indices.astype(jnp.int32)
    urow = (ui // BN) * G4 + ui % G4
    irow = (ii // BN) * G4 + ii % G4
    ju = ((ui // G4) % 4).reshape(B, 1)
    ji = ((ii // G4) % 4).reshape(B, 1)

    up = _repack_tc(ue_gmf.T, ue_mlp.T)
    upr = gather(urow.reshape(B // GCH, GCH), up)
    ip = _repack_tc(ie_gmf.T, ie_mlp.T)
    ipr = gather(irow.reshape(B // GCH, GCH), ip)

    h2 = W2.shape[1]
    out = _dense_tc(
        upr, ipr, ju, ji,
        W0[:D], W0[D:], b0.reshape(1, -1),
        W1, b1.reshape(1, -1),
        W2, b2.reshape(1, -1),
        Wp[:D].reshape(1, D), Wp[D:].reshape(1, h2), bp.reshape(1, 1),
    )
    return out.reshape(-1)


# speedup vs baseline: 4.3468x; 1.0024x over previous
import functools

import jax
import jax.numpy as jnp
from jax import lax
from jax.experimental import pallas as pl
from jax.experimental.pallas import tpu as pltpu
from jax.experimental.pallas import tpu_sc as plsc

B = 16384
D = 32
BN = 57344
G4 = BN // 4
NV = 1000000
NBLK = (NV + BN - 1) // BN
N4 = NBLK * G4
GCH = 128


def _repack_body(ta_ref, tb_ref, out_ref):
    xa = ta_ref[...].astype(jnp.bfloat16)
    xb = tb_ref[...].astype(jnp.bfloat16)
    lanes = jax.lax.broadcasted_iota(jnp.int32, (D, 128), 1)
    chans = jax.lax.broadcasted_iota(jnp.int32, (D, 128), 0)
    acc_a = jnp.zeros((G4, 128), jnp.float32)
    acc_b = jnp.zeros((G4, 128), jnp.float32)
    for j in range(4):
        ej = (lanes == j * D + chans).astype(jnp.bfloat16)
        acc_a = acc_a + jax.lax.dot_general(
            xa[:, j * G4:(j + 1) * G4], ej, (((0,), (0,)), ((), ())),
            preferred_element_type=jnp.float32)
        acc_b = acc_b + jax.lax.dot_general(
            xb[:, j * G4:(j + 1) * G4], ej, (((0,), (0,)), ((), ())),
            preferred_element_type=jnp.float32)
    wa = jax.lax.bitcast_convert_type(acc_a, jnp.uint32)
    wb = jax.lax.bitcast_convert_type(acc_b, jnp.uint32)
    out_ref[...] = wa | (wb >> 16)


def _repack_tc(ta_t, tb_t):
    return pl.pallas_call(
        _repack_body,
        grid=(NBLK,),
        in_specs=[
            pl.BlockSpec((D, BN), lambda i: (0, i)),
            pl.BlockSpec((D, BN), lambda i: (0, i)),
        ],
        out_specs=pl.BlockSpec((G4, 128), lambda i: (i, 0)),
        out_shape=jax.ShapeDtypeStruct((N4, 128), jnp.uint32),
    )(ta_t, tb_t)


def _build_sc_gather(nc, ns):
    nw = nc * ns
    bpw = B // nw
    nch = bpw // GCH
    mesh = plsc.VectorSubcoreMesh(core_axis_name="c", subcore_axis_name="s")

    @functools.partial(
        pl.kernel,
        mesh=mesh,
        compiler_params=pltpu.CompilerParams(use_tc_tiling_on_sc=False),
        out_type=jax.ShapeDtypeStruct((B, 128), jnp.uint32),
        scratch_types=[
            pltpu.VMEM((nch, GCH), jnp.int32),
            pltpu.VMEM((GCH, 128), jnp.uint32),
            pltpu.VMEM((GCH, 128), jnp.uint32),
            pltpu.SemaphoreType.DMA,
            pltpu.SemaphoreType.DMA,
        ],
    )
    def gather_kernel(idx_hbm, tab_hbm, out_hbm,
                      idx_v, rows_a, rows_b, sem_a, sem_b):
        wid = lax.axis_index("s") * nc + lax.axis_index("c")
        base = wid * bpw
        pltpu.sync_copy(idx_hbm.at[pl.ds(wid * nch, nch)], idx_v)

        bufs = (rows_a, rows_b)
        sems = (sem_a, sem_b)

        def start(s):
            pltpu.async_copy(tab_hbm.at[idx_v.at[s]], bufs[s % 2], sems[s % 2])

        start(0)
        for s in range(nch):
            p = s % 2
            if s + 1 < nch:
                start(s + 1)
            pltpu.make_async_copy(
                tab_hbm.at[idx_v.at[s]], bufs[p], sems[p]).wait()
            pltpu.sync_copy(bufs[p], out_hbm.at[pl.ds(base + s * GCH, GCH)])

    return gather_kernel


BM = 2048


def _dense_body(upw, ipw, ju, ji, w0u, w0i, b0, w1, b1, w2, b2,
                wpg, wpm, bp, out):
    uw = upw[...]
    iw = ipw[...]
    hi = jnp.uint32(0xFFFF0000)
    ugp = jax.lax.bitcast_convert_type(uw & hi, jnp.float32)
    ump = jax.lax.bitcast_convert_type(uw << 16, jnp.float32)
    igp = jax.lax.bitcast_convert_type(iw & hi, jnp.float32)
    imp = jax.lax.bitcast_convert_type(iw << 16, jnp.float32)
    sel_u = ju[...]
    sel_i = ji[...]
    ug = jnp.zeros((BM, D), jnp.float32)
    um = jnp.zeros((BM, D), jnp.float32)
    ig = jnp.zeros((BM, D), jnp.float32)
    im = jnp.zeros((BM, D), jnp.float32)
    for j in range(4):
        mu = sel_u == j
        mi = sel_i == j
        ug = ug + jnp.where(mu, ugp[:, j * D:(j + 1) * D], 0.0)
        um = um + jnp.where(mu, ump[:, j * D:(j + 1) * D], 0.0)
        ig = ig + jnp.where(mi, igp[:, j * D:(j + 1) * D], 0.0)
        im = im + jnp.where(mi, imp[:, j * D:(j + 1) * D], 0.0)
    h = jnp.dot(um, w0u[...], preferred_element_type=jnp.float32)
    h = h + jnp.dot(im, w0i[...], preferred_element_type=jnp.float32)
    h = jnp.maximum(h + b0[...], 0.0)
    h = jnp.maximum(jnp.dot(h, w1[...], preferred_element_type=jnp.float32) + b1[...], 0.0)
    h = jnp.maximum(jnp.dot(h, w2[...], preferred_element_type=jnp.float32) + b2[...], 0.0)
    g = ug * ig
    logit = (jnp.sum(g * wpg[...], axis=1, keepdims=True)
             + jnp.sum(h * wpm[...], axis=1, keepdims=True) + bp[...])
    out[...] = 1.0 / (1.0 + jnp.exp(-logit))


def _dense_tc(upw, ipw, ju, ji, w0u, w0i, b0, w1, b1, w2, b2, wpg, wpm, bp):
    row = lambda i: (i, 0)
    rep = lambda i: (0, 0)
    h0, h1, h2 = b0.shape[1], b1.shape[1], b2.shape[1]
    return pl.pallas_call(
        _dense_body,
        grid=(B // BM,),
        in_specs=[
            pl.BlockSpec((BM, 128), row),
            pl.BlockSpec((BM, 128), row),
            pl.BlockSpec((BM, 1), row),
            pl.BlockSpec((BM, 1), row),
            pl.BlockSpec((D, h0), rep),
            pl.BlockSpec((D, h0), rep),
            pl.BlockSpec((1, h0), rep),
            pl.BlockSpec((h0, h1), rep),
            pl.BlockSpec((1, h1), rep),
            pl.BlockSpec((h1, h2), rep),
            pl.BlockSpec((1, h2), rep),
            pl.BlockSpec((1, D), rep),
            pl.BlockSpec((1, h2), rep),
            pl.BlockSpec((1, 1), rep),
        ],
        out_specs=pl.BlockSpec((BM, 1), row),
        out_shape=jax.ShapeDtypeStruct((B, 1), jnp.float32),
    )(upw, ipw, ju, ji, w0u, w0i, b0, w1, b1, w2, b2, wpg, wpm, bp)


def kernel(user_indices, item_indices, ue_gmf, ie_gmf, ue_mlp, ie_mlp,
           W0, b0, W1, b1, W2, b2, Wp, bp):
    info = plsc.get_sparse_core_info()
    gather = _build_sc_gather(info.num_cores, info.num_subcores)

    ui = user_indices.astype(jnp.int32)
    ii = item_indices.astype(jnp.int32)
    urow = (ui // BN) * G4 + ui % G4
    irow = (ii // BN) * G4 + ii % G4
    ju = ((ui // G4) % 4).reshape(B, 1)
    ji = ((ii // G4) % 4).reshape(B, 1)

    up = _repack_tc(ue_gmf.T, ue_mlp.T)
    upr = gather(urow.reshape(B // GCH, GCH), up)
    ip = _repack_tc(ie_gmf.T, ie_mlp.T)
    ipr = gather(irow.reshape(B // GCH, GCH), ip)

    h2 = W2.shape[1]
    out = _dense_tc(
        upr, ipr, ju, ji,
        W0[:D], W0[D:], b0.reshape(1, -1),
        W1, b1.reshape(1, -1),
        W2, b2.reshape(1, -1),
        Wp[:D].reshape(1, D), Wp[D:].reshape(1, h2), bp.reshape(1, 1),
    )
    return out.reshape(-1)
